# baseline (speedup 1.0000x reference)
import functools

import jax
import jax.numpy as jnp
from jax import lax
from jax.experimental import pallas as pl
from jax.experimental.pallas import tpu as pltpu
from jax.experimental.pallas import tpu_sc as plsc

NUM_EMBEDDINGS = 1000000
EMBEDDING_DIM = 32
BATCH = 16384
N_FIELDS = 26

TOTAL = BATCH * N_FIELDS
NUM_CORES = 2
NUM_SUBCORES = 16
NW = NUM_CORES * NUM_SUBCORES
L = 16

W4 = NUM_EMBEDDINGS // 4
IDX_COLS = 128
IDX_ROWS = TOTAL // IDX_COLS
ROWS_PER_W = IDX_ROWS // NW
CHUNK = 128
N_CHUNKS = ROWS_PER_W
OUT_COLS = 128
OUT_ROWS = TOTAL * EMBEDDING_DIM // OUT_COLS
OUT_R_PER_CHUNK = CHUNK * EMBEDDING_DIM // OUT_COLS


@functools.partial(
    pl.kernel,
    out_type=jax.ShapeDtypeStruct((OUT_ROWS, OUT_COLS), jnp.float32),
    mesh=plsc.VectorSubcoreMesh(core_axis_name="c", subcore_axis_name="s"),
    scratch_types=[
        pltpu.VMEM((ROWS_PER_W, IDX_COLS), jnp.int32),
        pltpu.VMEM((ROWS_PER_W, IDX_COLS), jnp.int32),
        pltpu.VMEM((ROWS_PER_W, IDX_COLS), jnp.int32),
        pltpu.VMEM((2, CHUNK, 128), jnp.float32),
        pltpu.VMEM((2, OUT_R_PER_CHUNK, OUT_COLS), jnp.float32),
        pltpu.SemaphoreType.DMA,
        pltpu.SemaphoreType.DMA,
    ],
    compiler_params=pltpu.CompilerParams(
        use_tc_tiling_on_sc=True, needs_layout_passes=False),
)
def _gather_sc(table4, idx_hbm, out_hbm, idx_v, offs_v, sub_v, rows4_v,
               outb_v, sem_g, sem_s):
    wid = lax.axis_index("s") * NUM_CORES + lax.axis_index("c")
    base = wid * ROWS_PER_W

    pltpu.sync_copy(idx_hbm.at[pl.ds(base, ROWS_PER_W)], idx_v)

    def prep_row(r, _):
        for g in range(IDX_COLS // L):
            v = idx_v[r, pl.ds(g * L, L)]
            offs_v[r, pl.ds(g * L, L)] = lax.shift_right_logical(v, 2)
            sub_v[r, pl.ds(g * L, L)] = lax.shift_left(
                lax.bitwise_and(v, 3), 5)
        return 0

    lax.fori_loop(0, ROWS_PER_W, prep_row, 0)

    def issue_gather(i):
        pltpu.async_copy(
            table4.at[offs_v.at[i]], rows4_v.at[lax.rem(i, 2)], sem_g)

    def wait_gather():
        pltpu.make_async_copy(
            table4.at[offs_v.at[0]], rows4_v.at[0], sem_g).wait()

    def wait_store():
        pltpu.make_async_copy(
            outb_v.at[0], out_hbm.at[pl.ds(0, OUT_R_PER_CHUNK)], sem_s).wait()

    lanes = lax.iota(jnp.int32, L)

    def select(i, b):
        for g in range(CHUNK // L):
            krel = lanes + g * L
            subs = sub_v[i, pl.ds(g * L, L)]
            dst_flat = krel * EMBEDDING_DIM
            for c in range(EMBEDDING_DIM):
                vals = plsc.load_gather(rows4_v.at[b], [krel, subs + c])
                f = dst_flat + c
                plsc.store_scatter(
                    outb_v.at[b],
                    [lax.shift_right_logical(f, 7),
                     lax.bitwise_and(f, OUT_COLS - 1)], vals)

    issue_gather(0)

    def body(i, _):
        b = lax.rem(i, 2)
        wait_gather()

        @pl.when(i + 1 < N_CHUNKS)
        def _():
            issue_gather(i + 1)

        @pl.when(i >= 2)
        def _():
            wait_store()

        select(i, b)
        pltpu.async_copy(
            outb_v.at[b],
            out_hbm.at[pl.ds((base + i) * OUT_R_PER_CHUNK, OUT_R_PER_CHUNK)],
            sem_s)
        return 0

    lax.fori_loop(0, N_CHUNKS, body, 0)
    wait_store()
    wait_store()


def kernel(input, weight):
    table4 = weight.reshape(W4, 128)
    idx2 = input.reshape(IDX_ROWS, IDX_COLS)
    out = _gather_sc(table4, idx2)
    return out.reshape(BATCH, N_FIELDS, EMBEDDING_DIM)
